# baseline (speedup 1.0000x reference)
import functools

import jax
import jax.numpy as jnp
from jax import lax
from jax.experimental import pallas as pl
from jax.experimental.pallas import tpu as pltpu
from jax.experimental.pallas import tpu_sc as plsc

B = 4096
F = 26
D = 16
V = 100000
NUM = 13
H1, H2 = 512, 256

NC, NS = 2, 16
NW = NC * NS
SL = 32
BPW = B // NW
PW = BPW * SL
CH = 128
NCH = PW // CH


RPW = PW // 8


def _sc_gather_body(idx_hbm, idx16_hbm, emb_hbm, lr16_hbm, out_emb, out_lr,
                    idx_v, rows_v, sem):
    wid = lax.axis_index("s") * NC + lax.axis_index("c")

    def fire(tbl, out2d):
        def go(j, carry):
            pltpu.async_copy(tbl.at[idx_v.at[j]],
                             rows_v.at[pl.ds(j * CH, CH)], sem)
            return carry
        lax.fori_loop(0, NCH, go, 0)
        pltpu.make_async_copy(tbl.at[pl.ds(0, PW)], rows_v, sem).wait()
        for k in range(8):
            pltpu.sync_copy(
                rows_v.at[pl.ds(k * RPW, RPW)],
                out2d.at[pl.ds(wid * RPW, RPW), pl.ds(k * D, D)])

    pltpu.sync_copy(idx_hbm.at[wid], idx_v)
    fire(emb_hbm, out_emb)
    pltpu.sync_copy(idx16_hbm.at[wid], idx_v)
    fire(lr16_hbm, out_lr)


@functools.lru_cache(maxsize=None)
def _sc_gather():
    mesh = plsc.VectorSubcoreMesh(core_axis_name="c", subcore_axis_name="s")
    return pl.kernel(
        _sc_gather_body,
        mesh=mesh,
        compiler_params=pltpu.CompilerParams(use_tc_tiling_on_sc=False),
        out_type=(
            jax.ShapeDtypeStruct((B * 4, 128), jnp.float32),
            jax.ShapeDtypeStruct((B * 4, 128), jnp.float32),
        ),
        scratch_types=[
            pltpu.VMEM((NCH, CH), jnp.int32),
            pltpu.VMEM((PW, D), jnp.float32),
            pltpu.SemaphoreType.DMA,
        ],
    )


TB = 512
W = SL * D


def _dotT(x, w):
    return lax.dot_general(x, w, (((1,), (1,)), ((), ())),
                           precision=lax.Precision.HIGHEST,
                           preferred_element_type=jnp.float32)


def _dot(x, w):
    return lax.dot_general(x, w, (((1,), (0,)), ((), ())),
                           precision=lax.Precision.HIGHEST,
                           preferred_element_type=jnp.float32)


def _widen(x2):
    return x2.reshape(TB, W)


def _tc_body(flat4_ref, xnum_ref, lr4_ref, lane_ref, w1e_ref, w1n_ref, b1_ref,
             w2_ref, b2_ref, w3_ref, b3_ref, lrw_ref, lrb_ref, out_ref):
    lanes = lax.broadcasted_iota(jnp.int32, (TB, W), 1)
    valid = lanes < F * D
    flat = jnp.where(valid, _widen(flat4_ref[...]), 0.0)
    xnum = xnum_ref[...]
    h = _dotT(flat, w1e_ref[...]) + _dotT(xnum, w1n_ref[...]) + b1_ref[...]
    h = jnp.maximum(h, 0.0)
    h = jnp.maximum(_dotT(h, w2_ref[...]) + b2_ref[...], 0.0)
    dnn = jnp.sum(h * w3_ref[...], axis=1, keepdims=True) + b3_ref[0, 0]
    jj = lax.broadcasted_iota(jnp.int32, (W, D), 0)
    dd = lax.broadcasted_iota(jnp.int32, (W, D), 1)
    p = jnp.where(jj % D == dd, 1.0, 0.0).astype(jnp.float32)
    sum_emb = _dot(flat, p)
    sum_sq = jnp.sum(sum_emb * sum_emb, axis=1, keepdims=True)
    sq_sum = jnp.sum(flat * flat, axis=1, keepdims=True)
    fm = 0.5 * (sum_sq - sq_sum)
    lr512 = _widen(lr4_ref[...])
    ff = lax.broadcasted_iota(jnp.int32, (F, W), 0)
    cc = lax.broadcasted_iota(jnp.int32, (F, W), 1)
    e = jnp.where(cc // D == ff, 1.0, 0.0).astype(jnp.float32)
    lane_exp = _dot(lane_ref[...], e)
    mod16 = (lanes % D).astype(jnp.float32)
    sel = jnp.where((lane_exp == mod16) & valid, 1.0, 0.0)
    lr_sum = jnp.sum(lr512 * sel, axis=1, keepdims=True)
    lin = (lrb_ref[0, 0] + lr_sum
           + jnp.sum(xnum * lrw_ref[...], axis=1, keepdims=True))
    out_ref[...] = dnn + fm + lin


@functools.lru_cache(maxsize=None)
def _tc_call():
    grid = (B // TB,)
    row = lambda i: (i, 0)
    rep = lambda i: (0, 0)
    return pl.pallas_call(
        _tc_body,
        grid=grid,
        in_specs=[
            pl.BlockSpec((TB * 4, 128), row),
            pl.BlockSpec((TB, NUM), row),
            pl.BlockSpec((TB * 4, 128), row),
            pl.BlockSpec((TB, F), row),
            pl.BlockSpec((H1, W), rep),
            pl.BlockSpec((H1, NUM), rep),
            pl.BlockSpec((1, H1), rep),
            pl.BlockSpec((H2, H1), rep),
            pl.BlockSpec((1, H2), rep),
            pl.BlockSpec((1, H2), rep),
            pl.BlockSpec((1, 1), rep),
            pl.BlockSpec((1, NUM), rep),
            pl.BlockSpec((1, 1), rep),
        ],
        out_specs=pl.BlockSpec((TB, 1), row),
        out_shape=jax.ShapeDtypeStruct((B, 1), jnp.float32),
    )


def kernel(x_cat, x_num, emb_tables, lr_tables, lr_w, lr_bias,
           W1, b1, W2, b2, W3, b3):
    offs = (jnp.arange(F, dtype=jnp.int32) * V)[None, :]
    idx = x_cat.astype(jnp.int32) + offs
    def reorder(a):
        return (a.reshape(NW, BPW, 4, 8).transpose(0, 3, 1, 2)
                .reshape(NW, NCH, CH))

    idxp = reorder(jnp.pad(idx, ((0, 0), (0, SL - F))))
    idxp16 = reorder(jnp.pad(idx // D, ((0, 0), (0, SL - F))))
    lane_f = (idx % D).astype(jnp.float32)
    emb_flat = emb_tables.reshape(F * V, D)
    lr16_flat = lr_tables.reshape(F * V // D, D)
    flat4, lr4 = _sc_gather()(idxp, idxp16, emb_flat, lr16_flat)
    w1e = jnp.pad(W1[:, :F * D], ((0, 0), (0, W - F * D)))
    return _tc_call()(
        flat4, x_num, lr4, lane_f,
        w1e, W1[:, F * D:], b1.reshape(1, H1),
        W2, b2.reshape(1, H2),
        W3, b3.reshape(1, 1),
        lr_w, lr_bias.reshape(1, 1),
    )


# speedup vs baseline: 1.1952x; 1.1952x over previous
import functools

import jax
import jax.numpy as jnp
from jax import lax
from jax.experimental import pallas as pl
from jax.experimental.pallas import tpu as pltpu
from jax.experimental.pallas import tpu_sc as plsc

B = 4096
F = 26
D = 16
V = 100000
NUM = 13
H1, H2 = 512, 256

NC, NS = 2, 16
NW = NC * NS
SL = 32
BPW = B // NW
CH = 128


def _sc_gather_body(idx_hbm, idx16_hbm, emb_hbm, lr16_hbm, out_emb, out_lr,
                    idx_v, rows_v, rows128, sem):
    wid = lax.axis_index("s") * NC + lax.axis_index("c")

    def drain(n):
        def go(j, carry):
            pltpu.make_async_copy(
                lr16_hbm.at[pl.ds(0, CH)],
                rows_v.at[pl.ds(0, CH)], sem).wait()
            return carry
        lax.fori_loop(0, n, go, 0)

    def repack_and_copy(dst):
        def go(b, carry):
            for s in range(F):
                rows128[s // 8, b, pl.ds((s % 8) * D, D)] = (
                    rows_v[s * CH + b])
            return carry
        lax.fori_loop(0, BPW, go, 0)
        pltpu.sync_copy(rows128, dst.at[:, pl.ds(wid * BPW, BPW), :])

    pltpu.sync_copy(idx_hbm.at[wid], idx_v)

    def fire_emb(s, carry):
        pltpu.async_copy(emb_hbm.at[s].at[idx_v.at[s]],
                         rows_v.at[pl.ds(s * CH, CH)], sem)
        return carry
    lax.fori_loop(0, F, fire_emb, 0)
    drain(F)
    repack_and_copy(out_emb)

    pltpu.sync_copy(idx16_hbm.at[wid], idx_v)

    def fire_lr(s, carry):
        pltpu.async_copy(lr16_hbm.at[idx_v.at[s]],
                         rows_v.at[pl.ds(s * CH, CH)], sem)
        return carry
    lax.fori_loop(0, F, fire_lr, 0)
    drain(F)
    repack_and_copy(out_lr)


@functools.lru_cache(maxsize=None)
def _sc_gather():
    mesh = plsc.VectorSubcoreMesh(core_axis_name="c", subcore_axis_name="s")
    return pl.kernel(
        _sc_gather_body,
        mesh=mesh,
        compiler_params=pltpu.CompilerParams(use_tc_tiling_on_sc=False),
        out_type=(
            jax.ShapeDtypeStruct((4, B, 128), jnp.float32),
            jax.ShapeDtypeStruct((4, B, 128), jnp.float32),
        ),
        scratch_types=[
            pltpu.VMEM((F, CH), jnp.int32),
            pltpu.VMEM((F * CH, D), jnp.float32),
            pltpu.VMEM((4, BPW, 128), jnp.float32),
            pltpu.SemaphoreType.DMA,
        ],
    )


TB = 512
W = SL * D


def _dotT(x, w):
    return lax.dot_general(x, w, (((1,), (1,)), ((), ())),
                           precision=lax.Precision.HIGHEST,
                           preferred_element_type=jnp.float32)


def _dot(x, w):
    return lax.dot_general(x, w, (((1,), (0,)), ((), ())),
                           precision=lax.Precision.HIGHEST,
                           preferred_element_type=jnp.float32)


def _widen(x4):
    return jnp.concatenate([x4[0], x4[1], x4[2], x4[3]], axis=1)


def _tc_body(flat4_ref, xnum_ref, lr4_ref, lane_ref, w1e_ref, w1n_ref, b1_ref,
             w2_ref, b2_ref, w3_ref, b3_ref, lrw_ref, lrb_ref, out_ref):
    lanes = lax.broadcasted_iota(jnp.int32, (TB, W), 1)
    valid = lanes < F * D
    flat = jnp.where(valid, _widen(flat4_ref[...]), 0.0)
    xnum = xnum_ref[...]
    h = _dotT(flat, w1e_ref[...]) + _dotT(xnum, w1n_ref[...]) + b1_ref[...]
    h = jnp.maximum(h, 0.0)
    h = jnp.maximum(_dotT(h, w2_ref[...]) + b2_ref[...], 0.0)
    dnn = jnp.sum(h * w3_ref[...], axis=1, keepdims=True) + b3_ref[0, 0]
    jj = lax.broadcasted_iota(jnp.int32, (W, D), 0)
    dd = lax.broadcasted_iota(jnp.int32, (W, D), 1)
    p = jnp.where(jj % D == dd, 1.0, 0.0).astype(jnp.float32)
    sum_emb = _dot(flat, p)
    sum_sq = jnp.sum(sum_emb * sum_emb, axis=1, keepdims=True)
    sq_sum = jnp.sum(flat * flat, axis=1, keepdims=True)
    fm = 0.5 * (sum_sq - sq_sum)
    lr512 = jnp.where(valid, _widen(lr4_ref[...]), 0.0)
    ff = lax.broadcasted_iota(jnp.int32, (F, W), 0)
    cc = lax.broadcasted_iota(jnp.int32, (F, W), 1)
    e = jnp.where(cc // D == ff, 1.0, 0.0).astype(jnp.float32)
    lane_exp = _dot(lane_ref[...], e)
    mod16 = (lanes % D).astype(jnp.float32)
    sel = jnp.where((lane_exp == mod16) & valid, 1.0, 0.0)
    lr_sum = jnp.sum(lr512 * sel, axis=1, keepdims=True)
    lin = (lrb_ref[0, 0] + lr_sum
           + jnp.sum(xnum * lrw_ref[...], axis=1, keepdims=True))
    out_ref[...] = dnn + fm + lin


@functools.lru_cache(maxsize=None)
def _tc_call():
    grid = (B // TB,)
    row = lambda i: (i, 0)
    row3 = lambda i: (0, i, 0)
    rep = lambda i: (0, 0)
    return pl.pallas_call(
        _tc_body,
        grid=grid,
        in_specs=[
            pl.BlockSpec((4, TB, 128), row3),
            pl.BlockSpec((TB, NUM), row),
            pl.BlockSpec((4, TB, 128), row3),
            pl.BlockSpec((TB, F), row),
            pl.BlockSpec((H1, W), rep),
            pl.BlockSpec((H1, NUM), rep),
            pl.BlockSpec((1, H1), rep),
            pl.BlockSpec((H2, H1), rep),
            pl.BlockSpec((1, H2), rep),
            pl.BlockSpec((1, H2), rep),
            pl.BlockSpec((1, 1), rep),
            pl.BlockSpec((1, NUM), rep),
            pl.BlockSpec((1, 1), rep),
        ],
        out_specs=pl.BlockSpec((TB, 1), row),
        out_shape=jax.ShapeDtypeStruct((B, 1), jnp.float32),
    )


def kernel(x_cat, x_num, emb_tables, lr_tables, lr_w, lr_bias,
           W1, b1, W2, b2, W3, b3):
    xc = x_cat.astype(jnp.int32)
    offs = (jnp.arange(F, dtype=jnp.int32) * V)[None, :]
    lane_f = ((xc + offs) % D).astype(jnp.float32)

    def slot_major(a):
        return a.reshape(NW, BPW, F).transpose(0, 2, 1)

    idxf = slot_major(xc)
    idx16 = slot_major((xc + offs) // D)
    lr16_flat = lr_tables.reshape(F * V // D, D)
    flat4, lr4 = _sc_gather()(idxf, idx16, emb_tables, lr16_flat)
    w1e = jnp.pad(W1[:, :F * D], ((0, 0), (0, W - F * D)))
    return _tc_call()(
        flat4, x_num, lr4, lane_f,
        w1e, W1[:, F * D:], b1.reshape(1, H1),
        W2, b2.reshape(1, H2),
        W3, b3.reshape(1, 1),
        lr_w, lr_bias.reshape(1, 1),
    )
